# Initial kernel scaffold; baseline (speedup 1.0000x reference)
import jax
import jax.numpy as jnp
from jax.experimental import pallas as pl


def kernel(boxes, scores, features, W1, b1, Wc, bc, Wr, br):
    raise NotImplementedError("write your pallas kernel here")


# speedup vs baseline: 37.7441x; 37.7441x over previous
import functools

import jax
import jax.numpy as jnp
from jax import lax
from jax.experimental import pallas as pl
from jax.experimental.pallas import tpu as pltpu
from jax.experimental.pallas import tpu_sc as plsc

N = 5000
NP = 5120
B = 128
NB = NP // B
D = 1024
NUM_CLASSES = 81
MAX_OUT = 300
MP = 320
IOU_THRESH = 0.7

_HI = jax.lax.Precision.HIGHEST
_SC_PARAMS = pltpu.CompilerParams(needs_layout_passes=False)


def _rank_body(sr_ref, sc_ref, rank_ref):
    def row_block(a, _):
        sa = sr_ref[pl.ds(a, 1), :]
        ig = a * B + lax.broadcasted_iota(jnp.int32, (B, B), 1)

        def col_block(b, acc):
            sb = sc_ref[pl.ds(b * B, B), :]
            jg = b * B + lax.broadcasted_iota(jnp.int32, (B, B), 0)
            gt = (sb > sa).astype(jnp.int32)
            eq = jnp.logical_and(sb == sa, jg < ig).astype(jnp.int32)
            return acc + jnp.sum(gt + eq, axis=0, keepdims=True)

        acc = lax.fori_loop(0, NB, col_block, jnp.zeros((1, B), jnp.int32))
        rank_ref[pl.ds(a, 1), :] = acc
        return 0

    lax.fori_loop(0, NB, row_block, 0)


def _rank_call(sr, sc):
    return pl.pallas_call(
        _rank_body,
        out_shape=jax.ShapeDtypeStruct((NB, B), jnp.int32),
    )(sr, sc)


def _permute_body(rank_hbm, x1_hbm, y1_hbm, x2_hbm, y2_hbm,
                  order_hbm, sx1_hbm, sy1_hbm, sx2_hbm, sy2_hbm,
                  rank_v, x1_v, y1_v, x2_v, y2_v, order_v,
                  ox1_v, oy1_v, ox2_v, oy2_v):
    wid = lax.axis_index("s") * 2 + lax.axis_index("c")
    pltpu.sync_copy(rank_hbm, rank_v)
    pltpu.sync_copy(x1_hbm, x1_v)
    pltpu.sync_copy(y1_hbm, y1_v)
    pltpu.sync_copy(x2_hbm, x2_v)
    pltpu.sync_copy(y2_hbm, y2_v)
    lane = lax.iota(jnp.int32, 16)

    def scatter_step(i, _):
        idx = rank_v[pl.ds(i * 16, 16)]
        plsc.store_scatter(order_v, [idx], i * 16 + lane)
        return 0

    lax.fori_loop(0, NP // 16, scatter_step, 0)

    per = NP // 32
    base = wid * per

    def gather_step(i, _):
        sidx = order_v[pl.ds(base + i * 16, 16)]
        ox1_v[pl.ds(i * 16, 16)] = plsc.load_gather(x1_v, [sidx])
        oy1_v[pl.ds(i * 16, 16)] = plsc.load_gather(y1_v, [sidx])
        ox2_v[pl.ds(i * 16, 16)] = plsc.load_gather(x2_v, [sidx])
        oy2_v[pl.ds(i * 16, 16)] = plsc.load_gather(y2_v, [sidx])
        return 0

    lax.fori_loop(0, per // 16, gather_step, 0)
    pltpu.sync_copy(order_v.at[pl.ds(base, per)], order_hbm.at[pl.ds(base, per)])
    pltpu.sync_copy(ox1_v, sx1_hbm.at[pl.ds(base, per)])
    pltpu.sync_copy(oy1_v, sy1_hbm.at[pl.ds(base, per)])
    pltpu.sync_copy(ox2_v, sx2_hbm.at[pl.ds(base, per)])
    pltpu.sync_copy(oy2_v, sy2_hbm.at[pl.ds(base, per)])


def _permute_call(rank, x1, y1, x2, y2):
    per = NP // 32
    f32 = jnp.float32
    return pl.kernel(
        _permute_body,
        out_type=(
            jax.ShapeDtypeStruct((NP,), jnp.int32),
            jax.ShapeDtypeStruct((NP,), f32),
            jax.ShapeDtypeStruct((NP,), f32),
            jax.ShapeDtypeStruct((NP,), f32),
            jax.ShapeDtypeStruct((NP,), f32),
        ),
        mesh=plsc.VectorSubcoreMesh(core_axis_name="c", subcore_axis_name="s"),
        compiler_params=_SC_PARAMS,
        scratch_types=[
            pltpu.VMEM((NP,), jnp.int32),
            pltpu.VMEM((NP,), f32), pltpu.VMEM((NP,), f32),
            pltpu.VMEM((NP,), f32), pltpu.VMEM((NP,), f32),
            pltpu.VMEM((NP,), jnp.int32),
            pltpu.VMEM((per,), f32), pltpu.VMEM((per,), f32),
            pltpu.VMEM((per,), f32), pltpu.VMEM((per,), f32),
        ],
    )(rank, x1, y1, x2, y2)


def _tcol(ident, xrow):
    return lax.dot_general(ident, xrow, (((1,), (1,)), ((), ())),
                           preferred_element_type=jnp.float32, precision=_HI)


def _pair_overlap(x1c, y1c, x2c, y2c, ac, x1r, y1r, x2r, y2r, ar):
    ix1 = jnp.maximum(x1c, x1r)
    iy1 = jnp.maximum(y1c, y1r)
    ix2 = jnp.minimum(x2c, x2r)
    iy2 = jnp.minimum(y2c, y2r)
    inter = jnp.maximum(0.0, ix2 - ix1) * jnp.maximum(0.0, iy2 - iy1)
    union = ac + ar - inter
    iou = inter / (union + 1e-9)
    return (iou > IOU_THRESH).astype(jnp.float32)


def _nms_body(xr1_ref, yr1_ref, xr2_ref, yr2_ref,
              xc1_ref, yc1_ref, xc2_ref, yc2_ref,
              keep_ref, s_ref):
    s_ref[...] = jnp.zeros((NB, B), jnp.float32)
    ia = lax.broadcasted_iota(jnp.int32, (B, B), 0)
    ib = lax.broadcasted_iota(jnp.int32, (B, B), 1)
    ident = (ia == ib).astype(jnp.float32)
    uptri = (ia < ib).astype(jnp.float32)
    lane = lax.broadcasted_iota(jnp.int32, (1, B), 1)

    def block(k, _):
        x1r = xr1_ref[pl.ds(k, 1), :]
        y1r = yr1_ref[pl.ds(k, 1), :]
        x2r = xr2_ref[pl.ds(k, 1), :]
        y2r = yr2_ref[pl.ds(k, 1), :]
        x1c = xc1_ref[pl.ds(k * B, B), :]
        y1c = yc1_ref[pl.ds(k * B, B), :]
        x2c = xc2_ref[pl.ds(k * B, B), :]
        y2c = yc2_ref[pl.ds(k * B, B), :]
        ar = (x2r - x1r) * (y2r - y1r)
        ac = (x2c - x1c) * (y2c - y1c)

        okk = _pair_overlap(x1c, y1c, x2c, y2c, ac, x1r, y1r, x2r, y2r, ar) * uptri

        pad = (k * B + lane >= N).astype(jnp.float32)
        s0 = jnp.maximum(s_ref[pl.ds(k, 1), :], pad)

        def cond(state):
            srow, krow = state
            return jnp.sum((1.0 - srow) * (1.0 - krow)) > 0.0

        def body(state):
            srow, krow = state
            scol = _tcol(ident, srow)
            kcol = _tcol(ident, krow)
            supp = jnp.max(okk * kcol, axis=0, keepdims=True)
            srow = jnp.maximum(srow, supp)
            blocked = jnp.max(okk * (1.0 - scol), axis=0, keepdims=True)
            krow = jnp.maximum(krow, (1.0 - srow) * (1.0 - blocked))
            return srow, krow

        srow, krow = lax.while_loop(cond, body, (s0, jnp.zeros((1, B), jnp.float32)))
        keep_ref[pl.ds(k, 1), :] = krow
        kcol = _tcol(ident, krow)

        def push(p, _):
            px1 = xr1_ref[pl.ds(p, 1), :]
            py1 = yr1_ref[pl.ds(p, 1), :]
            px2 = xr2_ref[pl.ds(p, 1), :]
            py2 = yr2_ref[pl.ds(p, 1), :]
            pa = (px2 - px1) * (py2 - py1)
            okp = _pair_overlap(x1c, y1c, x2c, y2c, ac, px1, py1, px2, py2, pa)
            supp_p = jnp.max(okp * kcol, axis=0, keepdims=True)
            s_ref[pl.ds(p, 1), :] = jnp.maximum(s_ref[pl.ds(p, 1), :], supp_p)
            return 0

        lax.fori_loop(k + 1, NB, push, 0)
        return 0

    lax.fori_loop(0, NB, block, 0)


def _nms_call(sx1, sy1, sx2, sy2):
    r = lambda a: a.reshape(NB, B)
    c = lambda a: a.reshape(NP, 1)
    return pl.pallas_call(
        _nms_body,
        out_shape=jax.ShapeDtypeStruct((NB, B), jnp.float32),
        scratch_shapes=[pltpu.VMEM((NB, B), jnp.float32)],
    )(r(sx1), r(sy1), r(sx2), r(sy2), c(sx1), c(sy1), c(sx2), c(sy2))


def _compact_body(keep_hbm, order_hbm, x1_hbm, y1_hbm, x2_hbm, y2_hbm,
                  sel_hbm, vm_hbm, rx1_hbm, ry1_hbm, rx2_hbm, ry2_hbm,
                  keep_v, order_v, x1_v, y1_v, x2_v, y2_v,
                  pos_v, sel_v, vm_v, rx1_v, ry1_v, rx2_v, ry2_v):
    wid = lax.axis_index("s") * 2 + lax.axis_index("c")
    lane = lax.iota(jnp.int32, 16)

    @pl.when(wid == 0)
    def _():
        pltpu.sync_copy(keep_hbm, keep_v)
        pltpu.sync_copy(order_hbm, order_v)
        pltpu.sync_copy(x1_hbm, x1_v)
        pltpu.sync_copy(y1_hbm, y1_v)
        pltpu.sync_copy(x2_hbm, x2_v)
        pltpu.sync_copy(y2_hbm, y2_v)

        def zero_step(i, _):
            pos_v[pl.ds(i * 16, 16)] = jnp.zeros((16,), jnp.int32)
            return 0

        lax.fori_loop(0, MP // 16, zero_step, 0)

        def scan_step(i, carry):
            k16 = keep_v[pl.ds(i * 16, 16)]
            csum = plsc.cumsum(k16) + carry
            m = jnp.logical_and(k16 > 0, csum <= MAX_OUT)
            plsc.store_scatter(pos_v, [csum - 1], i * 16 + lane, mask=m)
            return carry + jnp.sum(k16)

        total = lax.fori_loop(0, NP // 16, scan_step, jnp.int32(0))

        def out_step(i, _):
            p16 = pos_v[pl.ds(i * 16, 16)]
            s16 = plsc.load_gather(order_v, [p16])
            sel_v[pl.ds(i * 16, 16)] = s16
            vmf = jnp.where(i * 16 + lane < total, 1.0, 0.0).astype(jnp.float32)
            vm_v[pl.ds(i * 16, 16)] = vmf
            rx1_v[pl.ds(i * 16, 16)] = plsc.load_gather(x1_v, [s16]) * vmf
            ry1_v[pl.ds(i * 16, 16)] = plsc.load_gather(y1_v, [s16]) * vmf
            rx2_v[pl.ds(i * 16, 16)] = plsc.load_gather(x2_v, [s16]) * vmf
            ry2_v[pl.ds(i * 16, 16)] = plsc.load_gather(y2_v, [s16]) * vmf
            return 0

        lax.fori_loop(0, MP // 16, out_step, 0)
        pltpu.sync_copy(sel_v, sel_hbm)
        pltpu.sync_copy(vm_v, vm_hbm)
        pltpu.sync_copy(rx1_v, rx1_hbm)
        pltpu.sync_copy(ry1_v, ry1_hbm)
        pltpu.sync_copy(rx2_v, rx2_hbm)
        pltpu.sync_copy(ry2_v, ry2_hbm)


def _compact_call(keep, order, x1, y1, x2, y2):
    f32 = jnp.float32
    i32 = jnp.int32
    return pl.kernel(
        _compact_body,
        out_type=(
            jax.ShapeDtypeStruct((MP,), i32),
            jax.ShapeDtypeStruct((MP,), f32),
            jax.ShapeDtypeStruct((MP,), f32),
            jax.ShapeDtypeStruct((MP,), f32),
            jax.ShapeDtypeStruct((MP,), f32),
            jax.ShapeDtypeStruct((MP,), f32),
        ),
        mesh=plsc.VectorSubcoreMesh(core_axis_name="c", subcore_axis_name="s"),
        compiler_params=_SC_PARAMS,
        scratch_types=[
            pltpu.VMEM((NP,), i32), pltpu.VMEM((NP,), i32),
            pltpu.VMEM((NP,), f32), pltpu.VMEM((NP,), f32),
            pltpu.VMEM((NP,), f32), pltpu.VMEM((NP,), f32),
            pltpu.VMEM((MP,), i32), pltpu.VMEM((MP,), i32),
            pltpu.VMEM((MP,), f32), pltpu.VMEM((MP,), f32),
            pltpu.VMEM((MP,), f32), pltpu.VMEM((MP,), f32),
            pltpu.VMEM((MP,), f32),
        ],
    )(keep, order, x1, y1, x2, y2)


def _featgather_body(feat_hbm, sel_hbm, out_hbm, sel_v, rows_v, sem):
    wid = lax.axis_index("s") * 2 + lax.axis_index("c")

    @pl.when(wid < MP // 16)
    def _():
        pltpu.sync_copy(sel_hbm.at[pl.ds(wid * 16, 16)], sel_v)
        pltpu.async_copy(feat_hbm.at[sel_v], rows_v, sem).wait()
        pltpu.sync_copy(rows_v, out_hbm.at[pl.ds(wid * 16, 16)])


def _featgather_call(features, sel):
    return pl.kernel(
        _featgather_body,
        out_type=jax.ShapeDtypeStruct((MP, D), jnp.float32),
        mesh=plsc.VectorSubcoreMesh(core_axis_name="c", subcore_axis_name="s"),
        compiler_params=_SC_PARAMS,
        scratch_types=[
            pltpu.VMEM((16,), jnp.int32),
            pltpu.VMEM((16, D), jnp.float32),
            pltpu.SemaphoreType.DMA,
        ],
    )(features, sel)


def _head_body(gf_ref, vm_ref, w1_ref, b1_ref, wcr_ref, bcr_ref,
               cls_ref, lin_ref):
    x = gf_ref[...] * vm_ref[...]
    h = jnp.maximum(
        lax.dot_general(x, w1_ref[...], (((1,), (0,)), ((), ())),
                        preferred_element_type=jnp.float32, precision=_HI)
        + b1_ref[...], 0.0)
    l = lax.dot_general(h, wcr_ref[...], (((1,), (0,)), ((), ())),
                        preferred_element_type=jnp.float32, precision=_HI) + bcr_ref[...]
    lane = lax.broadcasted_iota(jnp.int32, (MP, 128), 1)
    z = jnp.where(lane < NUM_CLASSES, l, -jnp.inf)
    zmax = jnp.max(z, axis=-1, keepdims=True)
    e = jnp.exp(z - zmax)
    cls_ref[...] = e / jnp.sum(e, axis=-1, keepdims=True)
    lin_ref[...] = l


def _head_call(gf, vm, W1, b1, Wcr, bcr):
    return pl.pallas_call(
        _head_body,
        out_shape=(
            jax.ShapeDtypeStruct((MP, 128), jnp.float32),
            jax.ShapeDtypeStruct((MP, 128), jnp.float32),
        ),
    )(gf, vm, W1, b1, Wcr, bcr)


def kernel(boxes, scores, features, W1, b1, Wc, bc, Wr, br):
    f32 = jnp.float32
    pad1 = lambda a, v: jnp.pad(a, (0, NP - N), constant_values=v)
    sp = pad1(scores.astype(f32), -1.0)
    x1 = pad1(boxes[:, 0], 0.0)
    y1 = pad1(boxes[:, 1], 0.0)
    x2 = pad1(boxes[:, 2], 0.0)
    y2 = pad1(boxes[:, 3], 0.0)

    rank = _rank_call(sp.reshape(NB, B), sp.reshape(NP, 1))
    order, sx1, sy1, sx2, sy2 = _permute_call(rank.reshape(NP), x1, y1, x2, y2)
    keep = _nms_call(sx1, sy1, sx2, sy2)
    sel, vm, rx1, ry1, rx2, ry2 = _compact_call(
        keep.reshape(NP).astype(jnp.int32), order, x1, y1, x2, y2)
    gf = _featgather_call(features, sel)

    Wcr = jnp.pad(jnp.concatenate([Wc, Wr], axis=1), ((0, 0), (0, 128 - NUM_CLASSES - 4)))
    bcr = jnp.pad(jnp.concatenate([bc, br]), (0, 128 - NUM_CLASSES - 4)).reshape(1, 128)
    cls, lin = _head_call(gf, vm.reshape(MP, 1), W1, b1.reshape(1, D), Wcr, bcr)

    class_scores = cls[:MAX_OUT, :NUM_CLASSES]
    bbox_deltas = lin[:MAX_OUT, NUM_CLASSES:NUM_CLASSES + 4]
    rois = jnp.stack([rx1[:MAX_OUT], ry1[:MAX_OUT], rx2[:MAX_OUT], ry2[:MAX_OUT]], axis=1)
    return class_scores, bbox_deltas, rois


# speedup vs baseline: 64.6276x; 1.7123x over previous
import functools

import jax
import jax.numpy as jnp
from jax import lax
from jax.experimental import pallas as pl
from jax.experimental.pallas import tpu as pltpu
from jax.experimental.pallas import tpu_sc as plsc

N = 5000
NP = 5120
B = 128
NB = NP // B
D = 1024
NUM_CLASSES = 81
MAX_OUT = 300
MP = 384
IOU_THRESH = 0.7

_HI = jax.lax.Precision.HIGHEST
_SC_PARAMS = pltpu.CompilerParams(needs_layout_passes=False)


def _rank_body(sr_ref, rank_ref):
    ia = lax.broadcasted_iota(jnp.int32, (B, B), 0)
    ib = lax.broadcasted_iota(jnp.int32, (B, B), 1)
    ident = (ia == ib).astype(jnp.float32)

    def row_block(a, _):
        sa = sr_ref[:, pl.ds(a * B, B)]
        sac = _tcol(ident, sa)

        def col_block(q, acc):
            for u in range(4):
                b = q * 4 + u
                sb = sr_ref[:, pl.ds(b * B, B)]
                acc = acc + jnp.where(b < a, (sb >= sac).astype(jnp.float32),
                                      (sb > sac).astype(jnp.float32))
            return acc

        acc = lax.fori_loop(0, NB // 4, col_block, jnp.zeros((B, B), jnp.float32))
        tie = jnp.logical_and(sa == sac, ib < ia).astype(jnp.float32)
        rank = jnp.sum(acc + tie, axis=1, keepdims=True)
        rank_ref[pl.ds(a * B, B), :] = rank.astype(jnp.int32)
        return 0

    lax.fori_loop(0, NB, row_block, 0)


def _rank_call(sp):
    return pl.pallas_call(
        _rank_body,
        out_shape=jax.ShapeDtypeStruct((NP, 1), jnp.int32),
    )(sp.reshape(1, NP))


def _permute_body(rank_hbm, x1_hbm, y1_hbm, x2_hbm, y2_hbm,
                  order_hbm, sx1_hbm, sy1_hbm, sx2_hbm, sy2_hbm,
                  rank_v, x1_v, y1_v, x2_v, y2_v, order_v,
                  ox1_v, oy1_v, ox2_v, oy2_v):
    wid = lax.axis_index("s") * 2 + lax.axis_index("c")
    pltpu.sync_copy(rank_hbm, rank_v)
    pltpu.sync_copy(x1_hbm, x1_v)
    pltpu.sync_copy(y1_hbm, y1_v)
    pltpu.sync_copy(x2_hbm, x2_v)
    pltpu.sync_copy(y2_hbm, y2_v)
    lane = lax.iota(jnp.int32, 16)

    def scatter_step(i, _):
        idx = rank_v[pl.ds(i * 16, 16)]
        plsc.store_scatter(order_v, [idx], i * 16 + lane)
        return 0

    lax.fori_loop(0, NP // 16, scatter_step, 0)

    per = NP // 32
    base = wid * per

    def gather_step(i, _):
        sidx = order_v[pl.ds(base + i * 16, 16)]
        ox1_v[pl.ds(i * 16, 16)] = plsc.load_gather(x1_v, [sidx])
        oy1_v[pl.ds(i * 16, 16)] = plsc.load_gather(y1_v, [sidx])
        ox2_v[pl.ds(i * 16, 16)] = plsc.load_gather(x2_v, [sidx])
        oy2_v[pl.ds(i * 16, 16)] = plsc.load_gather(y2_v, [sidx])
        return 0

    lax.fori_loop(0, per // 16, gather_step, 0)
    pltpu.sync_copy(order_v.at[pl.ds(base, per)], order_hbm.at[pl.ds(base, per)])
    pltpu.sync_copy(ox1_v, sx1_hbm.at[pl.ds(base, per)])
    pltpu.sync_copy(oy1_v, sy1_hbm.at[pl.ds(base, per)])
    pltpu.sync_copy(ox2_v, sx2_hbm.at[pl.ds(base, per)])
    pltpu.sync_copy(oy2_v, sy2_hbm.at[pl.ds(base, per)])


def _permute_call(rank, x1, y1, x2, y2):
    per = NP // 32
    f32 = jnp.float32
    return pl.kernel(
        _permute_body,
        out_type=(
            jax.ShapeDtypeStruct((NP,), jnp.int32),
            jax.ShapeDtypeStruct((NP,), f32),
            jax.ShapeDtypeStruct((NP,), f32),
            jax.ShapeDtypeStruct((NP,), f32),
            jax.ShapeDtypeStruct((NP,), f32),
        ),
        mesh=plsc.VectorSubcoreMesh(core_axis_name="c", subcore_axis_name="s"),
        compiler_params=_SC_PARAMS,
        scratch_types=[
            pltpu.VMEM((NP,), jnp.int32),
            pltpu.VMEM((NP,), f32), pltpu.VMEM((NP,), f32),
            pltpu.VMEM((NP,), f32), pltpu.VMEM((NP,), f32),
            pltpu.VMEM((NP,), jnp.int32),
            pltpu.VMEM((per,), f32), pltpu.VMEM((per,), f32),
            pltpu.VMEM((per,), f32), pltpu.VMEM((per,), f32),
        ],
    )(rank, x1, y1, x2, y2)


def _tcol(ident, xrow):
    return lax.dot_general(ident, xrow, (((1,), (1,)), ((), ())),
                           preferred_element_type=jnp.float32, precision=_HI)


def _pair_overlap(x1c, y1c, x2c, y2c, ac, x1r, y1r, x2r, y2r, ar):
    ix1 = jnp.maximum(x1c, x1r)
    iy1 = jnp.maximum(y1c, y1r)
    ix2 = jnp.minimum(x2c, x2r)
    iy2 = jnp.minimum(y2c, y2r)
    inter = jnp.maximum(0.0, ix2 - ix1) * jnp.maximum(0.0, iy2 - iy1)
    union = ac + ar - inter
    iou = inter / (union + 1e-9)
    return (iou > IOU_THRESH).astype(jnp.float32)


CH = 1024
NCH = NP // CH


def _nms_body(xr1_ref, yr1_ref, xr2_ref, yr2_ref, keep_ref, s_ref):
    s_ref[...] = jnp.zeros((1, NP), jnp.float32)
    ia = lax.broadcasted_iota(jnp.int32, (B, B), 0)
    ib = lax.broadcasted_iota(jnp.int32, (B, B), 1)
    ident = (ia == ib).astype(jnp.float32)
    uptri = (ia < ib).astype(jnp.float32)
    lane = lax.broadcasted_iota(jnp.int32, (1, B), 1)
    lane_ch = lax.broadcasted_iota(jnp.int32, (1, CH), 1)

    def block(k, _):
        x1r = xr1_ref[:, pl.ds(k * B, B)]
        y1r = yr1_ref[:, pl.ds(k * B, B)]
        x2r = xr2_ref[:, pl.ds(k * B, B)]
        y2r = yr2_ref[:, pl.ds(k * B, B)]
        x1c = _tcol(ident, x1r)
        y1c = _tcol(ident, y1r)
        x2c = _tcol(ident, x2r)
        y2c = _tcol(ident, y2r)
        ar = (x2r - x1r) * (y2r - y1r)
        ac = (x2c - x1c) * (y2c - y1c)

        base = _pair_overlap(x1c, y1c, x2c, y2c, ac, x1r, y1r, x2r, y2r, ar)
        okk = base * uptri
        okt = base * (ia > ib).astype(jnp.float32)

        pad = (k * B + lane >= N).astype(jnp.float32)
        s0 = jnp.maximum(s_ref[:, pl.ds(k * B, B)], pad)
        s0c = _tcol(ident, s0)

        def cond(state):
            srow, _, krow, _ = state
            return jnp.sum((1.0 - srow) * (1.0 - krow)) > 0.0

        def body(state):
            srow, scol, krow, kcol = state
            supp_r = jnp.max(okk * kcol, axis=0, keepdims=True)
            supp_c = jnp.max(okt * krow, axis=1, keepdims=True)
            srow2 = jnp.maximum(srow, supp_r)
            scol2 = jnp.maximum(scol, supp_c)
            blk_r = jnp.max(okk * (1.0 - scol), axis=0, keepdims=True)
            blk_c = jnp.max(okt * (1.0 - srow), axis=1, keepdims=True)
            krow2 = jnp.maximum(krow, (1.0 - srow2) * (1.0 - blk_r))
            kcol2 = jnp.maximum(kcol, (1.0 - scol2) * (1.0 - blk_c))
            return srow2, scol2, krow2, kcol2

        srow, scol, krow, kcol = lax.while_loop(
            cond, body,
            (s0, s0c, jnp.zeros((1, B), jnp.float32), jnp.zeros((B, 1), jnp.float32)))
        keep_ref[:, pl.ds(k * B, B)] = krow

        def push(c, _):
            px1 = xr1_ref[:, pl.ds(c * CH, CH)]
            py1 = yr1_ref[:, pl.ds(c * CH, CH)]
            px2 = xr2_ref[:, pl.ds(c * CH, CH)]
            py2 = yr2_ref[:, pl.ds(c * CH, CH)]
            pa = (px2 - px1) * (py2 - py1)
            okp = _pair_overlap(x1c, y1c, x2c, y2c, ac, px1, py1, px2, py2, pa)
            supp_p = jnp.max(okp * kcol, axis=0, keepdims=True)
            m = (c * CH + lane_ch >= (k + 1) * B).astype(jnp.float32)
            s_ref[:, pl.ds(c * CH, CH)] = jnp.maximum(
                s_ref[:, pl.ds(c * CH, CH)], supp_p * m)
            return 0

        lax.fori_loop((k + 1) * B // CH, NCH, push, 0)
        return 0

    lax.fori_loop(0, NB, block, 0)


def _nms_call(sx1, sy1, sx2, sy2):
    r = lambda a: a.reshape(1, NP)
    return pl.pallas_call(
        _nms_body,
        out_shape=jax.ShapeDtypeStruct((1, NP), jnp.float32),
        scratch_shapes=[pltpu.VMEM((1, NP), jnp.float32)],
    )(r(sx1), r(sy1), r(sx2), r(sy2))


RPT = MP // 16


def _compact_body(keep_hbm, order_hbm, x1_hbm, y1_hbm, x2_hbm, y2_hbm, feat_hbm,
                  vm_hbm, rx1_hbm, ry1_hbm, rx2_hbm, ry2_hbm, gf_hbm,
                  keep_v, order_v, x1_v, y1_v, x2_v, y2_v,
                  pos_v, sel_v, vm_v, rx1_v, ry1_v, rx2_v, ry2_v,
                  sel_sh, sel_t, rows_v, sem):
    cid = lax.axis_index("c")
    sid = lax.axis_index("s")
    lane = lax.iota(jnp.int32, 16)

    @pl.when(cid == 0)
    def _():
        @pl.when(sid == 0)
        def _():
            pltpu.sync_copy(keep_hbm, keep_v)
            pltpu.sync_copy(order_hbm, order_v)
            pltpu.sync_copy(x1_hbm, x1_v)
            pltpu.sync_copy(y1_hbm, y1_v)
            pltpu.sync_copy(x2_hbm, x2_v)
            pltpu.sync_copy(y2_hbm, y2_v)

            def zero_step(i, _):
                pos_v[pl.ds(i * 16, 16)] = jnp.zeros((16,), jnp.int32)
                return 0

            lax.fori_loop(0, MP // 16, zero_step, 0)

            def scan_step(i, carry):
                k16 = keep_v[pl.ds(i * 16, 16)]
                csum = plsc.cumsum(k16) + carry
                m = jnp.logical_and(k16 > 0, csum <= MAX_OUT)
                plsc.store_scatter(pos_v, [csum - 1], i * 16 + lane, mask=m)
                return carry + jnp.sum(k16)

            total = lax.fori_loop(0, NP // 16, scan_step, jnp.int32(0))

            def out_step(i, _):
                p16 = pos_v[pl.ds(i * 16, 16)]
                s16 = plsc.load_gather(order_v, [p16])
                sel_v[pl.ds(i * 16, 16)] = s16
                vmf = jnp.where(i * 16 + lane < total, 1.0, 0.0).astype(jnp.float32)
                vm_v[pl.ds(i * 16, 16)] = vmf
                rx1_v[pl.ds(i * 16, 16)] = plsc.load_gather(x1_v, [s16]) * vmf
                ry1_v[pl.ds(i * 16, 16)] = plsc.load_gather(y1_v, [s16]) * vmf
                rx2_v[pl.ds(i * 16, 16)] = plsc.load_gather(x2_v, [s16]) * vmf
                ry2_v[pl.ds(i * 16, 16)] = plsc.load_gather(y2_v, [s16]) * vmf
                return 0

            lax.fori_loop(0, MP // 16, out_step, 0)
            pltpu.sync_copy(vm_v, vm_hbm)
            pltpu.sync_copy(rx1_v, rx1_hbm)
            pltpu.sync_copy(ry1_v, ry1_hbm)
            pltpu.sync_copy(rx2_v, rx2_hbm)
            pltpu.sync_copy(ry2_v, ry2_hbm)
            pltpu.sync_copy(sel_v, sel_sh)

        plsc.subcore_barrier()
        pltpu.sync_copy(sel_sh.at[pl.ds(sid * RPT, RPT)], sel_t)
        pltpu.async_copy(feat_hbm.at[sel_t], rows_v, sem).wait()
        pltpu.sync_copy(rows_v, gf_hbm.at[pl.ds(sid * RPT, RPT)])


def _compact_call(keep, order, x1, y1, x2, y2, features):
    f32 = jnp.float32
    i32 = jnp.int32
    return pl.kernel(
        _compact_body,
        out_type=(
            jax.ShapeDtypeStruct((MP,), f32),
            jax.ShapeDtypeStruct((MP,), f32),
            jax.ShapeDtypeStruct((MP,), f32),
            jax.ShapeDtypeStruct((MP,), f32),
            jax.ShapeDtypeStruct((MP,), f32),
            jax.ShapeDtypeStruct((MP, D), f32),
        ),
        mesh=plsc.VectorSubcoreMesh(core_axis_name="c", subcore_axis_name="s"),
        compiler_params=_SC_PARAMS,
        scratch_types=[
            pltpu.VMEM((NP,), i32), pltpu.VMEM((NP,), i32),
            pltpu.VMEM((NP,), f32), pltpu.VMEM((NP,), f32),
            pltpu.VMEM((NP,), f32), pltpu.VMEM((NP,), f32),
            pltpu.VMEM((MP,), i32), pltpu.VMEM((MP,), i32),
            pltpu.VMEM((MP,), f32), pltpu.VMEM((MP,), f32),
            pltpu.VMEM((MP,), f32), pltpu.VMEM((MP,), f32),
            pltpu.VMEM((MP,), f32),
            pltpu.VMEM_SHARED((MP,), i32),
            pltpu.VMEM((RPT,), i32),
            pltpu.VMEM((RPT, D), f32),
            pltpu.SemaphoreType.DMA,
        ],
    )(keep, order, x1, y1, x2, y2, features)


def _head_body(gf_ref, vm_ref, w1_ref, b1_ref, wcr_ref, bcr_ref,
               cls_ref, lin_ref):
    x = gf_ref[...] * vm_ref[...]
    h = jnp.maximum(
        lax.dot_general(x, w1_ref[...], (((1,), (0,)), ((), ())),
                        preferred_element_type=jnp.float32, precision=_HI)
        + b1_ref[...], 0.0)
    l = lax.dot_general(h, wcr_ref[...], (((1,), (0,)), ((), ())),
                        preferred_element_type=jnp.float32, precision=_HI) + bcr_ref[...]
    lane = lax.broadcasted_iota(jnp.int32, (MP, 128), 1)
    z = jnp.where(lane < NUM_CLASSES, l, -jnp.inf)
    zmax = jnp.max(z, axis=-1, keepdims=True)
    e = jnp.exp(z - zmax)
    cls_ref[...] = e / jnp.sum(e, axis=-1, keepdims=True)
    lin_ref[...] = l


def _head_call(gf, vm, W1, b1, Wcr, bcr):
    return pl.pallas_call(
        _head_body,
        out_shape=(
            jax.ShapeDtypeStruct((MP, 128), jnp.float32),
            jax.ShapeDtypeStruct((MP, 128), jnp.float32),
        ),
    )(gf, vm, W1, b1, Wcr, bcr)


def kernel(boxes, scores, features, W1, b1, Wc, bc, Wr, br):
    f32 = jnp.float32
    pad1 = lambda a, v: jnp.pad(a, (0, NP - N), constant_values=v)
    sp = pad1(scores.astype(f32), -1.0)
    x1 = pad1(boxes[:, 0], 0.0)
    y1 = pad1(boxes[:, 1], 0.0)
    x2 = pad1(boxes[:, 2], 0.0)
    y2 = pad1(boxes[:, 3], 0.0)

    rank = _rank_call(sp)
    order, sx1, sy1, sx2, sy2 = _permute_call(rank.reshape(NP), x1, y1, x2, y2)
    keep = _nms_call(sx1, sy1, sx2, sy2)
    vm, rx1, ry1, rx2, ry2, gf = _compact_call(
        keep.reshape(NP).astype(jnp.int32), order, x1, y1, x2, y2, features)

    Wcr = jnp.pad(jnp.concatenate([Wc, Wr], axis=1), ((0, 0), (0, 128 - NUM_CLASSES - 4)))
    bcr = jnp.pad(jnp.concatenate([bc, br]), (0, 128 - NUM_CLASSES - 4)).reshape(1, 128)
    cls, lin = _head_call(gf, vm.reshape(MP, 1), W1, b1.reshape(1, D), Wcr, bcr)

    class_scores = cls[:MAX_OUT, :NUM_CLASSES]
    bbox_deltas = lin[:MAX_OUT, NUM_CLASSES:NUM_CLASSES + 4]
    rois = jnp.stack([rx1[:MAX_OUT], ry1[:MAX_OUT], rx2[:MAX_OUT], ry2[:MAX_OUT]], axis=1)
    return class_scores, bbox_deltas, rois


# speedup vs baseline: 67.9668x; 1.0517x over previous
import functools

import jax
import jax.numpy as jnp
from jax import lax
from jax.experimental import pallas as pl
from jax.experimental.pallas import tpu as pltpu
from jax.experimental.pallas import tpu_sc as plsc

N = 5000
NP = 5120
B = 128
NB = NP // B
D = 1024
NUM_CLASSES = 81
MAX_OUT = 300
MP = 384
IOU_THRESH = 0.7

_HI = jax.lax.Precision.HIGHEST
_SC_PARAMS = pltpu.CompilerParams(needs_layout_passes=False)


def _rank_body(sr_ref, rank_ref):
    ia = lax.broadcasted_iota(jnp.int32, (B, B), 0)
    ib = lax.broadcasted_iota(jnp.int32, (B, B), 1)
    ident = (ia == ib).astype(jnp.float32)

    def row_block(a, _):
        sa = sr_ref[:, pl.ds(a * B, B)]
        sac = _tcol(ident, sa)

        def col_block(q, acc):
            for u in range(4):
                b = q * 4 + u
                sb = sr_ref[:, pl.ds(b * B, B)]
                acc = acc + jnp.where(b < a, (sb >= sac).astype(jnp.float32),
                                      (sb > sac).astype(jnp.float32))
            return acc

        acc = lax.fori_loop(0, NB // 4, col_block, jnp.zeros((B, B), jnp.float32))
        tie = jnp.logical_and(sa == sac, ib < ia).astype(jnp.float32)
        rank = jnp.sum(acc + tie, axis=1, keepdims=True)
        rank_ref[pl.ds(a * B, B), :] = rank.astype(jnp.int32)
        return 0

    lax.fori_loop(0, NB, row_block, 0)


def _rank_call(sp):
    return pl.pallas_call(
        _rank_body,
        out_shape=jax.ShapeDtypeStruct((NP, 1), jnp.int32),
    )(sp.reshape(1, NP))


def _permute_body(rank_hbm, x1_hbm, y1_hbm, x2_hbm, y2_hbm,
                  order_hbm, sx1_hbm, sy1_hbm, sx2_hbm, sy2_hbm,
                  rank_v, x1_v, y1_v, x2_v, y2_v, order_v,
                  ox1_v, oy1_v, ox2_v, oy2_v):
    wid = lax.axis_index("s") * 2 + lax.axis_index("c")
    pltpu.sync_copy(rank_hbm, rank_v)
    pltpu.sync_copy(x1_hbm, x1_v)
    pltpu.sync_copy(y1_hbm, y1_v)
    pltpu.sync_copy(x2_hbm, x2_v)
    pltpu.sync_copy(y2_hbm, y2_v)
    lane = lax.iota(jnp.int32, 16)

    def scatter_step(q, _):
        for u in range(4):
            i = q * 4 + u
            idx = rank_v[pl.ds(i * 16, 16)]
            plsc.store_scatter(order_v, [idx], i * 16 + lane)
        return 0

    lax.fori_loop(0, NP // 64, scatter_step, 0)

    per = NP // 32
    base = wid * per

    def gather_step(i, _):
        sidx = order_v[pl.ds(base + i * 16, 16)]
        ox1_v[pl.ds(i * 16, 16)] = plsc.load_gather(x1_v, [sidx])
        oy1_v[pl.ds(i * 16, 16)] = plsc.load_gather(y1_v, [sidx])
        ox2_v[pl.ds(i * 16, 16)] = plsc.load_gather(x2_v, [sidx])
        oy2_v[pl.ds(i * 16, 16)] = plsc.load_gather(y2_v, [sidx])
        return 0

    lax.fori_loop(0, per // 16, gather_step, 0)
    pltpu.sync_copy(order_v.at[pl.ds(base, per)], order_hbm.at[pl.ds(base, per)])
    pltpu.sync_copy(ox1_v, sx1_hbm.at[pl.ds(base, per)])
    pltpu.sync_copy(oy1_v, sy1_hbm.at[pl.ds(base, per)])
    pltpu.sync_copy(ox2_v, sx2_hbm.at[pl.ds(base, per)])
    pltpu.sync_copy(oy2_v, sy2_hbm.at[pl.ds(base, per)])


def _permute_call(rank, x1, y1, x2, y2):
    per = NP // 32
    f32 = jnp.float32
    return pl.kernel(
        _permute_body,
        out_type=(
            jax.ShapeDtypeStruct((NP,), jnp.int32),
            jax.ShapeDtypeStruct((NP,), f32),
            jax.ShapeDtypeStruct((NP,), f32),
            jax.ShapeDtypeStruct((NP,), f32),
            jax.ShapeDtypeStruct((NP,), f32),
        ),
        mesh=plsc.VectorSubcoreMesh(core_axis_name="c", subcore_axis_name="s"),
        compiler_params=_SC_PARAMS,
        scratch_types=[
            pltpu.VMEM((NP,), jnp.int32),
            pltpu.VMEM((NP,), f32), pltpu.VMEM((NP,), f32),
            pltpu.VMEM((NP,), f32), pltpu.VMEM((NP,), f32),
            pltpu.VMEM((NP,), jnp.int32),
            pltpu.VMEM((per,), f32), pltpu.VMEM((per,), f32),
            pltpu.VMEM((per,), f32), pltpu.VMEM((per,), f32),
        ],
    )(rank, x1, y1, x2, y2)


def _tcol(ident, xrow):
    return lax.dot_general(ident, xrow, (((1,), (1,)), ((), ())),
                           preferred_element_type=jnp.float32, precision=_HI)


def _pair_overlap(x1c, y1c, x2c, y2c, ac, x1r, y1r, x2r, y2r, ar):
    ix1 = jnp.maximum(x1c, x1r)
    iy1 = jnp.maximum(y1c, y1r)
    ix2 = jnp.minimum(x2c, x2r)
    iy2 = jnp.minimum(y2c, y2r)
    inter = jnp.maximum(0.0, ix2 - ix1) * jnp.maximum(0.0, iy2 - iy1)
    union = ac + ar - inter
    iou = inter / (union + 1e-9)
    return (iou > IOU_THRESH).astype(jnp.float32)


CH = 1024
NCH = NP // CH


def _nms_body(xr1_ref, yr1_ref, xr2_ref, yr2_ref, keep_ref, s_ref):
    s_ref[...] = jnp.zeros((1, NP), jnp.float32)
    ia = lax.broadcasted_iota(jnp.int32, (B, B), 0)
    ib = lax.broadcasted_iota(jnp.int32, (B, B), 1)
    ident = (ia == ib).astype(jnp.float32)
    uptri = (ia < ib).astype(jnp.float32)
    lane = lax.broadcasted_iota(jnp.int32, (1, B), 1)
    lane_ch = lax.broadcasted_iota(jnp.int32, (1, CH), 1)

    def block(k, _):
        x1r = xr1_ref[:, pl.ds(k * B, B)]
        y1r = yr1_ref[:, pl.ds(k * B, B)]
        x2r = xr2_ref[:, pl.ds(k * B, B)]
        y2r = yr2_ref[:, pl.ds(k * B, B)]
        x1c = _tcol(ident, x1r)
        y1c = _tcol(ident, y1r)
        x2c = _tcol(ident, x2r)
        y2c = _tcol(ident, y2r)
        ar = (x2r - x1r) * (y2r - y1r)
        ac = (x2c - x1c) * (y2c - y1c)

        base = _pair_overlap(x1c, y1c, x2c, y2c, ac, x1r, y1r, x2r, y2r, ar)
        okk = base * uptri
        okt = base * (ia > ib).astype(jnp.float32)

        pad = (k * B + lane >= N).astype(jnp.float32)
        s0 = jnp.maximum(s_ref[:, pl.ds(k * B, B)], pad)
        s0c = _tcol(ident, s0)

        def cond(state):
            srow, _, krow, _ = state
            return jnp.sum((1.0 - srow) * (1.0 - krow)) > 0.0

        def body(state):
            srow, scol, krow, kcol = state
            supp_r = jnp.max(okk * kcol, axis=0, keepdims=True)
            supp_c = jnp.max(okt * krow, axis=1, keepdims=True)
            srow2 = jnp.maximum(srow, supp_r)
            scol2 = jnp.maximum(scol, supp_c)
            blk_r = jnp.max(okk * (1.0 - scol), axis=0, keepdims=True)
            blk_c = jnp.max(okt * (1.0 - srow), axis=1, keepdims=True)
            krow2 = jnp.maximum(krow, (1.0 - srow2) * (1.0 - blk_r))
            kcol2 = jnp.maximum(kcol, (1.0 - scol2) * (1.0 - blk_c))
            return srow2, scol2, krow2, kcol2

        srow, scol, krow, kcol = lax.while_loop(
            cond, body,
            (s0, s0c, jnp.zeros((1, B), jnp.float32), jnp.zeros((B, 1), jnp.float32)))
        keep_ref[:, pl.ds(k * B, B)] = krow

        def push(c, _):
            px1 = xr1_ref[:, pl.ds(c * CH, CH)]
            py1 = yr1_ref[:, pl.ds(c * CH, CH)]
            px2 = xr2_ref[:, pl.ds(c * CH, CH)]
            py2 = yr2_ref[:, pl.ds(c * CH, CH)]
            pa = (px2 - px1) * (py2 - py1)
            okp = _pair_overlap(x1c, y1c, x2c, y2c, ac, px1, py1, px2, py2, pa)
            supp_p = jnp.max(okp * kcol, axis=0, keepdims=True)
            m = (c * CH + lane_ch >= (k + 1) * B).astype(jnp.float32)
            s_ref[:, pl.ds(c * CH, CH)] = jnp.maximum(
                s_ref[:, pl.ds(c * CH, CH)], supp_p * m)
            return 0

        lax.fori_loop((k + 1) * B // CH, NCH, push, 0)
        return 0

    lax.fori_loop(0, NB, block, 0)


def _nms_call(sx1, sy1, sx2, sy2):
    r = lambda a: a.reshape(1, NP)
    return pl.pallas_call(
        _nms_body,
        out_shape=jax.ShapeDtypeStruct((1, NP), jnp.float32),
        scratch_shapes=[pltpu.VMEM((1, NP), jnp.float32)],
    )(r(sx1), r(sy1), r(sx2), r(sy2))


RPT = MP // 16


def _compact_body(keep_hbm, order_hbm, x1_hbm, y1_hbm, x2_hbm, y2_hbm, feat_hbm,
                  vm_hbm, rx1_hbm, ry1_hbm, rx2_hbm, ry2_hbm, gf_hbm,
                  keep_v, order_v, x1_v, y1_v, x2_v, y2_v,
                  pos_v, sel_v, vm_v, rx1_v, ry1_v, rx2_v, ry2_v,
                  sel_sh, sel_t, rows_v, sem):
    cid = lax.axis_index("c")
    sid = lax.axis_index("s")
    lane = lax.iota(jnp.int32, 16)

    @pl.when(cid == 0)
    def _():
        @pl.when(sid == 0)
        def _():
            pltpu.sync_copy(keep_hbm, keep_v)
            pltpu.sync_copy(order_hbm, order_v)
            pltpu.sync_copy(x1_hbm, x1_v)
            pltpu.sync_copy(y1_hbm, y1_v)
            pltpu.sync_copy(x2_hbm, x2_v)
            pltpu.sync_copy(y2_hbm, y2_v)

            def zero_step(i, _):
                pos_v[pl.ds(i * 16, 16)] = jnp.zeros((16,), jnp.int32)
                return 0

            lax.fori_loop(0, MP // 16, zero_step, 0)

            def scan_step(q, carry):
                for u in range(4):
                    i = q * 4 + u
                    k16 = keep_v[pl.ds(i * 16, 16)]
                    csum = plsc.cumsum(k16) + carry
                    m = jnp.logical_and(k16 > 0, csum <= MAX_OUT)
                    plsc.store_scatter(pos_v, [csum - 1], i * 16 + lane, mask=m)
                    carry = carry + jnp.sum(k16)
                return carry

            total = lax.fori_loop(0, NP // 64, scan_step, jnp.int32(0))

            def out_step(i, _):
                p16 = pos_v[pl.ds(i * 16, 16)]
                s16 = plsc.load_gather(order_v, [p16])
                sel_v[pl.ds(i * 16, 16)] = s16
                vmf = jnp.where(i * 16 + lane < total, 1.0, 0.0).astype(jnp.float32)
                vm_v[pl.ds(i * 16, 16)] = vmf
                rx1_v[pl.ds(i * 16, 16)] = plsc.load_gather(x1_v, [s16]) * vmf
                ry1_v[pl.ds(i * 16, 16)] = plsc.load_gather(y1_v, [s16]) * vmf
                rx2_v[pl.ds(i * 16, 16)] = plsc.load_gather(x2_v, [s16]) * vmf
                ry2_v[pl.ds(i * 16, 16)] = plsc.load_gather(y2_v, [s16]) * vmf
                return 0

            lax.fori_loop(0, MP // 16, out_step, 0)
            pltpu.sync_copy(vm_v, vm_hbm)
            pltpu.sync_copy(rx1_v, rx1_hbm)
            pltpu.sync_copy(ry1_v, ry1_hbm)
            pltpu.sync_copy(rx2_v, rx2_hbm)
            pltpu.sync_copy(ry2_v, ry2_hbm)
            pltpu.sync_copy(sel_v, sel_sh)

        plsc.subcore_barrier()
        pltpu.sync_copy(sel_sh.at[pl.ds(sid * RPT, RPT)], sel_t)
        pltpu.async_copy(feat_hbm.at[sel_t], rows_v, sem).wait()
        pltpu.sync_copy(rows_v, gf_hbm.at[pl.ds(sid * RPT, RPT)])


def _compact_call(keep, order, x1, y1, x2, y2, features):
    f32 = jnp.float32
    i32 = jnp.int32
    return pl.kernel(
        _compact_body,
        out_type=(
            jax.ShapeDtypeStruct((MP,), f32),
            jax.ShapeDtypeStruct((MP,), f32),
            jax.ShapeDtypeStruct((MP,), f32),
            jax.ShapeDtypeStruct((MP,), f32),
            jax.ShapeDtypeStruct((MP,), f32),
            jax.ShapeDtypeStruct((MP, D), f32),
        ),
        mesh=plsc.VectorSubcoreMesh(core_axis_name="c", subcore_axis_name="s"),
        compiler_params=_SC_PARAMS,
        scratch_types=[
            pltpu.VMEM((NP,), i32), pltpu.VMEM((NP,), i32),
            pltpu.VMEM((NP,), f32), pltpu.VMEM((NP,), f32),
            pltpu.VMEM((NP,), f32), pltpu.VMEM((NP,), f32),
            pltpu.VMEM((MP,), i32), pltpu.VMEM((MP,), i32),
            pltpu.VMEM((MP,), f32), pltpu.VMEM((MP,), f32),
            pltpu.VMEM((MP,), f32), pltpu.VMEM((MP,), f32),
            pltpu.VMEM((MP,), f32),
            pltpu.VMEM_SHARED((MP,), i32),
            pltpu.VMEM((RPT,), i32),
            pltpu.VMEM((RPT, D), f32),
            pltpu.SemaphoreType.DMA,
        ],
    )(keep, order, x1, y1, x2, y2, features)


def _head_body(gf_ref, vm_ref, w1_ref, b1_ref, wcr_ref, bcr_ref,
               cls_ref, lin_ref):
    x = gf_ref[...] * vm_ref[...]
    hi = jax.lax.Precision.DEFAULT
    h = jnp.maximum(
        lax.dot_general(x, w1_ref[...], (((1,), (0,)), ((), ())),
                        preferred_element_type=jnp.float32, precision=hi)
        + b1_ref[...], 0.0)
    l = lax.dot_general(h, wcr_ref[...], (((1,), (0,)), ((), ())),
                        preferred_element_type=jnp.float32, precision=hi) + bcr_ref[...]
    lane = lax.broadcasted_iota(jnp.int32, (MP, 128), 1)
    z = jnp.where(lane < NUM_CLASSES, l, -jnp.inf)
    zmax = jnp.max(z, axis=-1, keepdims=True)
    e = jnp.exp(z - zmax)
    cls_ref[...] = e / jnp.sum(e, axis=-1, keepdims=True)
    lin_ref[...] = l


def _head_call(gf, vm, W1, b1, Wcr, bcr):
    return pl.pallas_call(
        _head_body,
        out_shape=(
            jax.ShapeDtypeStruct((MP, 128), jnp.float32),
            jax.ShapeDtypeStruct((MP, 128), jnp.float32),
        ),
    )(gf, vm, W1, b1, Wcr, bcr)


def kernel(boxes, scores, features, W1, b1, Wc, bc, Wr, br):
    f32 = jnp.float32
    pad1 = lambda a, v: jnp.pad(a, (0, NP - N), constant_values=v)
    sp = pad1(scores.astype(f32), -1.0)
    x1 = pad1(boxes[:, 0], 0.0)
    y1 = pad1(boxes[:, 1], 0.0)
    x2 = pad1(boxes[:, 2], 0.0)
    y2 = pad1(boxes[:, 3], 0.0)

    rank = _rank_call(sp)
    order, sx1, sy1, sx2, sy2 = _permute_call(rank.reshape(NP), x1, y1, x2, y2)
    keep = _nms_call(sx1, sy1, sx2, sy2)
    vm, rx1, ry1, rx2, ry2, gf = _compact_call(
        keep.reshape(NP).astype(jnp.int32), order, x1, y1, x2, y2, features)

    Wcr = jnp.pad(jnp.concatenate([Wc, Wr], axis=1), ((0, 0), (0, 128 - NUM_CLASSES - 4)))
    bcr = jnp.pad(jnp.concatenate([bc, br]), (0, 128 - NUM_CLASSES - 4)).reshape(1, 128)
    cls, lin = _head_call(gf, vm.reshape(MP, 1), W1, b1.reshape(1, D), Wcr, bcr)

    class_scores = cls[:MAX_OUT, :NUM_CLASSES]
    bbox_deltas = lin[:MAX_OUT, NUM_CLASSES:NUM_CLASSES + 4]
    rois = jnp.stack([rx1[:MAX_OUT], ry1[:MAX_OUT], rx2[:MAX_OUT], ry2[:MAX_OUT]], axis=1)
    return class_scores, bbox_deltas, rois


# speedup vs baseline: 67.9727x; 1.0001x over previous
import jax
import jax.numpy as jnp
from jax import lax
from jax.experimental import pallas as pl
from jax.experimental.pallas import tpu as pltpu
from jax.experimental.pallas import tpu_sc as plsc

N = 5000
NP = 5120
B = 128
NB = NP // B
D = 1024
NUM_CLASSES = 81
MAX_OUT = 300
MP = 384
IOU_THRESH = 0.7

_HI = jax.lax.Precision.HIGHEST
_SC_PARAMS = pltpu.CompilerParams(needs_layout_passes=False)


def _rank_body(sr_ref, rank_ref):
    ia = lax.broadcasted_iota(jnp.int32, (B, B), 0)
    ib = lax.broadcasted_iota(jnp.int32, (B, B), 1)
    ident = (ia == ib).astype(jnp.float32)

    def row_block(a, _):
        sa = sr_ref[:, pl.ds(a * B, B)]
        sac = _tcol(ident, sa)

        def col_block(q, acc):
            for u in range(4):
                b = q * 4 + u
                sb = sr_ref[:, pl.ds(b * B, B)]
                acc = acc + jnp.where(b < a, (sb >= sac).astype(jnp.float32),
                                      (sb > sac).astype(jnp.float32))
            return acc

        acc = lax.fori_loop(0, NB // 4, col_block, jnp.zeros((B, B), jnp.float32))
        tie = jnp.logical_and(sa == sac, ib < ia).astype(jnp.float32)
        rank = jnp.sum(acc + tie, axis=1, keepdims=True)
        rank_ref[pl.ds(a * B, B), :] = rank.astype(jnp.int32)
        return 0

    lax.fori_loop(0, NB, row_block, 0)


def _rank_call(sp):
    return pl.pallas_call(
        _rank_body,
        out_shape=jax.ShapeDtypeStruct((NP, 1), jnp.int32),
    )(sp.reshape(1, NP))


def _permute_body(rank_hbm, x1_hbm, y1_hbm, x2_hbm, y2_hbm,
                  order_hbm, sx1_hbm, sy1_hbm, sx2_hbm, sy2_hbm,
                  rank_v, x1_v, y1_v, x2_v, y2_v, order_v,
                  ox1_v, oy1_v, ox2_v, oy2_v):
    wid = lax.axis_index("s") * 2 + lax.axis_index("c")
    pltpu.sync_copy(rank_hbm, rank_v)
    pltpu.sync_copy(x1_hbm, x1_v)
    pltpu.sync_copy(y1_hbm, y1_v)
    pltpu.sync_copy(x2_hbm, x2_v)
    pltpu.sync_copy(y2_hbm, y2_v)
    lane = lax.iota(jnp.int32, 16)

    def scatter_step(q, _):
        for u in range(4):
            i = q * 4 + u
            idx = rank_v[pl.ds(i * 16, 16)]
            plsc.store_scatter(order_v, [idx], i * 16 + lane)
        return 0

    lax.fori_loop(0, NP // 64, scatter_step, 0)

    per = NP // 32
    base = wid * per

    def gather_step(i, _):
        sidx = order_v[pl.ds(base + i * 16, 16)]
        ox1_v[pl.ds(i * 16, 16)] = plsc.load_gather(x1_v, [sidx])
        oy1_v[pl.ds(i * 16, 16)] = plsc.load_gather(y1_v, [sidx])
        ox2_v[pl.ds(i * 16, 16)] = plsc.load_gather(x2_v, [sidx])
        oy2_v[pl.ds(i * 16, 16)] = plsc.load_gather(y2_v, [sidx])
        return 0

    lax.fori_loop(0, per // 16, gather_step, 0)
    pltpu.sync_copy(order_v.at[pl.ds(base, per)], order_hbm.at[pl.ds(base, per)])
    pltpu.sync_copy(ox1_v, sx1_hbm.at[pl.ds(base, per)])
    pltpu.sync_copy(oy1_v, sy1_hbm.at[pl.ds(base, per)])
    pltpu.sync_copy(ox2_v, sx2_hbm.at[pl.ds(base, per)])
    pltpu.sync_copy(oy2_v, sy2_hbm.at[pl.ds(base, per)])


def _permute_call(rank, x1, y1, x2, y2):
    per = NP // 32
    f32 = jnp.float32
    return pl.kernel(
        _permute_body,
        out_type=(
            jax.ShapeDtypeStruct((NP,), jnp.int32),
            jax.ShapeDtypeStruct((NP,), f32),
            jax.ShapeDtypeStruct((NP,), f32),
            jax.ShapeDtypeStruct((NP,), f32),
            jax.ShapeDtypeStruct((NP,), f32),
        ),
        mesh=plsc.VectorSubcoreMesh(core_axis_name="c", subcore_axis_name="s"),
        compiler_params=_SC_PARAMS,
        scratch_types=[
            pltpu.VMEM((NP,), jnp.int32),
            pltpu.VMEM((NP,), f32), pltpu.VMEM((NP,), f32),
            pltpu.VMEM((NP,), f32), pltpu.VMEM((NP,), f32),
            pltpu.VMEM((NP,), jnp.int32),
            pltpu.VMEM((per,), f32), pltpu.VMEM((per,), f32),
            pltpu.VMEM((per,), f32), pltpu.VMEM((per,), f32),
        ],
    )(rank, x1, y1, x2, y2)


def _tcol(ident, xrow):
    return lax.dot_general(ident, xrow, (((1,), (1,)), ((), ())),
                           preferred_element_type=jnp.float32, precision=_HI)


def _pair_overlap(x1c, y1c, x2c, y2c, ac, x1r, y1r, x2r, y2r, ar):
    ix1 = jnp.maximum(x1c, x1r)
    iy1 = jnp.maximum(y1c, y1r)
    ix2 = jnp.minimum(x2c, x2r)
    iy2 = jnp.minimum(y2c, y2r)
    inter = jnp.maximum(0.0, ix2 - ix1) * jnp.maximum(0.0, iy2 - iy1)
    union = ac + ar - inter
    iou = inter / (union + 1e-9)
    return (iou > IOU_THRESH).astype(jnp.float32)


CH = 1024
NCH = NP // CH


def _nms_body(xr1_ref, yr1_ref, xr2_ref, yr2_ref, keep_ref, s_ref):
    s_ref[...] = jnp.zeros((1, NP), jnp.float32)
    ia = lax.broadcasted_iota(jnp.int32, (B, B), 0)
    ib = lax.broadcasted_iota(jnp.int32, (B, B), 1)
    ident = (ia == ib).astype(jnp.float32)
    uptri = (ia < ib).astype(jnp.float32)
    lane = lax.broadcasted_iota(jnp.int32, (1, B), 1)
    lane_ch = lax.broadcasted_iota(jnp.int32, (1, CH), 1)

    def block(k, _):
        x1r = xr1_ref[:, pl.ds(k * B, B)]
        y1r = yr1_ref[:, pl.ds(k * B, B)]
        x2r = xr2_ref[:, pl.ds(k * B, B)]
        y2r = yr2_ref[:, pl.ds(k * B, B)]
        x1c = _tcol(ident, x1r)
        y1c = _tcol(ident, y1r)
        x2c = _tcol(ident, x2r)
        y2c = _tcol(ident, y2r)
        ar = (x2r - x1r) * (y2r - y1r)
        ac = (x2c - x1c) * (y2c - y1c)

        base = _pair_overlap(x1c, y1c, x2c, y2c, ac, x1r, y1r, x2r, y2r, ar)
        okk = base * uptri
        okt = base * (ia > ib).astype(jnp.float32)

        pad = (k * B + lane >= N).astype(jnp.float32)
        s0 = jnp.maximum(s_ref[:, pl.ds(k * B, B)], pad)
        s0c = _tcol(ident, s0)

        def cond(state):
            srow, _, krow, _ = state
            return jnp.sum((1.0 - srow) * (1.0 - krow)) > 0.0

        def body(state):
            srow, scol, krow, kcol = state
            supp_r = jnp.max(okk * kcol, axis=0, keepdims=True)
            supp_c = jnp.max(okt * krow, axis=1, keepdims=True)
            srow2 = jnp.maximum(srow, supp_r)
            scol2 = jnp.maximum(scol, supp_c)
            blk_r = jnp.max(okk * (1.0 - scol), axis=0, keepdims=True)
            blk_c = jnp.max(okt * (1.0 - srow), axis=1, keepdims=True)
            krow2 = jnp.maximum(krow, (1.0 - srow2) * (1.0 - blk_r))
            kcol2 = jnp.maximum(kcol, (1.0 - scol2) * (1.0 - blk_c))
            return srow2, scol2, krow2, kcol2

        srow, scol, krow, kcol = lax.while_loop(
            cond, body,
            (s0, s0c, jnp.zeros((1, B), jnp.float32), jnp.zeros((B, 1), jnp.float32)))
        keep_ref[:, pl.ds(k * B, B)] = krow

        def push(c, _):
            px1 = xr1_ref[:, pl.ds(c * CH, CH)]
            py1 = yr1_ref[:, pl.ds(c * CH, CH)]
            px2 = xr2_ref[:, pl.ds(c * CH, CH)]
            py2 = yr2_ref[:, pl.ds(c * CH, CH)]
            pa = (px2 - px1) * (py2 - py1)
            okp = _pair_overlap(x1c, y1c, x2c, y2c, ac, px1, py1, px2, py2, pa)
            supp_p = jnp.max(okp * kcol, axis=0, keepdims=True)
            m = (c * CH + lane_ch >= (k + 1) * B).astype(jnp.float32)
            s_ref[:, pl.ds(c * CH, CH)] = jnp.maximum(
                s_ref[:, pl.ds(c * CH, CH)], supp_p * m)
            return 0

        lax.fori_loop((k + 1) * B // CH, NCH, push, 0)
        return 0

    lax.fori_loop(0, NB, block, 0)


def _nms_call(sx1, sy1, sx2, sy2):
    r = lambda a: a.reshape(1, NP)
    return pl.pallas_call(
        _nms_body,
        out_shape=jax.ShapeDtypeStruct((1, NP), jnp.float32),
        scratch_shapes=[pltpu.VMEM((1, NP), jnp.float32)],
    )(r(sx1), r(sy1), r(sx2), r(sy2))


RPT = MP // 16


def _compact_body(keep_hbm, order_hbm, x1_hbm, y1_hbm, x2_hbm, y2_hbm, feat_hbm,
                  vm_hbm, rx1_hbm, ry1_hbm, rx2_hbm, ry2_hbm, gf_hbm,
                  keep_v, order_v, x1_v, y1_v, x2_v, y2_v,
                  pos_v, sel_v, vm_v, rx1_v, ry1_v, rx2_v, ry2_v,
                  sel_sh, sel_t, rows_v, sem):
    cid = lax.axis_index("c")
    sid = lax.axis_index("s")
    lane = lax.iota(jnp.int32, 16)

    @pl.when(cid == 0)
    def _():
        @pl.when(sid == 0)
        def _():
            pltpu.sync_copy(keep_hbm, keep_v)
            pltpu.sync_copy(order_hbm, order_v)
            pltpu.sync_copy(x1_hbm, x1_v)
            pltpu.sync_copy(y1_hbm, y1_v)
            pltpu.sync_copy(x2_hbm, x2_v)
            pltpu.sync_copy(y2_hbm, y2_v)

            def zero_step(i, _):
                pos_v[pl.ds(i * 16, 16)] = jnp.zeros((16,), jnp.int32)
                return 0

            lax.fori_loop(0, MP // 16, zero_step, 0)

            def scan_step(q, carry):
                for u in range(4):
                    i = q * 4 + u
                    k16 = keep_v[pl.ds(i * 16, 16)]
                    csum = plsc.cumsum(k16) + carry
                    m = jnp.logical_and(k16 > 0, csum <= MAX_OUT)
                    plsc.store_scatter(pos_v, [csum - 1], i * 16 + lane, mask=m)
                    carry = carry + jnp.sum(k16)
                return carry

            total = lax.fori_loop(0, NP // 64, scan_step, jnp.int32(0))

            def out_step(i, _):
                p16 = pos_v[pl.ds(i * 16, 16)]
                s16 = plsc.load_gather(order_v, [p16])
                sel_v[pl.ds(i * 16, 16)] = s16
                vmf = jnp.where(i * 16 + lane < total, 1.0, 0.0).astype(jnp.float32)
                vm_v[pl.ds(i * 16, 16)] = vmf
                rx1_v[pl.ds(i * 16, 16)] = plsc.load_gather(x1_v, [s16]) * vmf
                ry1_v[pl.ds(i * 16, 16)] = plsc.load_gather(y1_v, [s16]) * vmf
                rx2_v[pl.ds(i * 16, 16)] = plsc.load_gather(x2_v, [s16]) * vmf
                ry2_v[pl.ds(i * 16, 16)] = plsc.load_gather(y2_v, [s16]) * vmf
                return 0

            lax.fori_loop(0, MP // 16, out_step, 0)
            pltpu.sync_copy(vm_v, vm_hbm)
            pltpu.sync_copy(rx1_v, rx1_hbm)
            pltpu.sync_copy(ry1_v, ry1_hbm)
            pltpu.sync_copy(rx2_v, rx2_hbm)
            pltpu.sync_copy(ry2_v, ry2_hbm)
            pltpu.sync_copy(sel_v, sel_sh)

        plsc.subcore_barrier()
        pltpu.sync_copy(sel_sh.at[pl.ds(sid * RPT, RPT)], sel_t)
        pltpu.async_copy(feat_hbm.at[sel_t], rows_v, sem).wait()
        pltpu.sync_copy(rows_v, gf_hbm.at[pl.ds(sid * RPT, RPT)])


def _compact_call(keep, order, x1, y1, x2, y2, features):
    f32 = jnp.float32
    i32 = jnp.int32
    return pl.kernel(
        _compact_body,
        out_type=(
            jax.ShapeDtypeStruct((MP,), f32),
            jax.ShapeDtypeStruct((MP,), f32),
            jax.ShapeDtypeStruct((MP,), f32),
            jax.ShapeDtypeStruct((MP,), f32),
            jax.ShapeDtypeStruct((MP,), f32),
            jax.ShapeDtypeStruct((MP, D), f32),
        ),
        mesh=plsc.VectorSubcoreMesh(core_axis_name="c", subcore_axis_name="s"),
        compiler_params=_SC_PARAMS,
        scratch_types=[
            pltpu.VMEM((NP,), i32), pltpu.VMEM((NP,), i32),
            pltpu.VMEM((NP,), f32), pltpu.VMEM((NP,), f32),
            pltpu.VMEM((NP,), f32), pltpu.VMEM((NP,), f32),
            pltpu.VMEM((MP,), i32), pltpu.VMEM((MP,), i32),
            pltpu.VMEM((MP,), f32), pltpu.VMEM((MP,), f32),
            pltpu.VMEM((MP,), f32), pltpu.VMEM((MP,), f32),
            pltpu.VMEM((MP,), f32),
            pltpu.VMEM_SHARED((MP,), i32),
            pltpu.VMEM((RPT,), i32),
            pltpu.VMEM((RPT, D), f32),
            pltpu.SemaphoreType.DMA,
        ],
    )(keep, order, x1, y1, x2, y2, features)


def _head_body(gf_ref, vm_ref, w1_ref, b1_ref, wcr_ref, bcr_ref,
               cls_ref, lin_ref):
    x = gf_ref[...] * vm_ref[...]
    hi = jax.lax.Precision.DEFAULT
    h = jnp.maximum(
        lax.dot_general(x, w1_ref[...], (((1,), (0,)), ((), ())),
                        preferred_element_type=jnp.float32, precision=hi)
        + b1_ref[...], 0.0)
    l = lax.dot_general(h, wcr_ref[...], (((1,), (0,)), ((), ())),
                        preferred_element_type=jnp.float32, precision=hi) + bcr_ref[...]
    lane = lax.broadcasted_iota(jnp.int32, (MP, 128), 1)
    z = jnp.where(lane < NUM_CLASSES, l, -jnp.inf)
    zmax = jnp.max(z, axis=-1, keepdims=True)
    e = jnp.exp(z - zmax)
    cls_ref[...] = e / jnp.sum(e, axis=-1, keepdims=True)
    lin_ref[...] = l


def _head_call(gf, vm, W1, b1, Wcr, bcr):
    return pl.pallas_call(
        _head_body,
        out_shape=(
            jax.ShapeDtypeStruct((MP, 128), jnp.float32),
            jax.ShapeDtypeStruct((MP, 128), jnp.float32),
        ),
    )(gf, vm, W1, b1, Wcr, bcr)


def kernel(boxes, scores, features, W1, b1, Wc, bc, Wr, br):
    f32 = jnp.float32
    pad1 = lambda a, v: jnp.pad(a, (0, NP - N), constant_values=v)
    sp = pad1(scores.astype(f32), -1.0)
    x1 = pad1(boxes[:, 0], 0.0)
    y1 = pad1(boxes[:, 1], 0.0)
    x2 = pad1(boxes[:, 2], 0.0)
    y2 = pad1(boxes[:, 3], 0.0)

    rank = _rank_call(sp)
    order, sx1, sy1, sx2, sy2 = _permute_call(rank.reshape(NP), x1, y1, x2, y2)
    keep = _nms_call(sx1, sy1, sx2, sy2)
    vm, rx1, ry1, rx2, ry2, gf = _compact_call(
        keep.reshape(NP).astype(jnp.int32), order, x1, y1, x2, y2, features)

    Wcr = jnp.pad(jnp.concatenate([Wc, Wr], axis=1), ((0, 0), (0, 128 - NUM_CLASSES - 4)))
    bcr = jnp.pad(jnp.concatenate([bc, br]), (0, 128 - NUM_CLASSES - 4)).reshape(1, 128)
    cls, lin = _head_call(gf, vm.reshape(MP, 1), W1, b1.reshape(1, D), Wcr, bcr)

    class_scores = cls[:MAX_OUT, :NUM_CLASSES]
    bbox_deltas = lin[:MAX_OUT, NUM_CLASSES:NUM_CLASSES + 4]
    rois = jnp.stack([rx1[:MAX_OUT], ry1[:MAX_OUT], rx2[:MAX_OUT], ry2[:MAX_OUT]], axis=1)
    return class_scores, bbox_deltas, rois
